# baseline (speedup 1.0000x reference)
import functools

import jax
import jax.numpy as jnp
from jax.experimental import pallas as pl

N_ROWS = 16384
C_IN = 7
C_OUT = 6
BLK = 8192
GRID = N_ROWS // BLK


def _deepjet_body(x_ref, o_ref):
    x = x_ref[...]
    c = x[3:4, :]
    cvl = x[5:6, :]
    qg = x[6:7, :]
    t = c / cvl - c
    o_ref[0:4, :] = x[0:4, :]
    o_ref[4:5, :] = (1.0 - qg) * t
    o_ref[5:6, :] = qg * t


@jax.jit
def _deepjet(xt):
    return pl.pallas_call(
        _deepjet_body,
        grid=(GRID,),
        in_specs=[pl.BlockSpec((C_IN, BLK), lambda i: (0, i))],
        out_specs=pl.BlockSpec((C_OUT, BLK), lambda i: (0, i)),
        out_shape=jax.ShapeDtypeStruct((C_OUT, N_ROWS), jnp.float32),
    )(xt)


def kernel(x):
    return _deepjet(x.T).T


# speedup vs baseline: 1.0320x; 1.0320x over previous
import functools

import jax
import jax.numpy as jnp
from jax.experimental import pallas as pl

N_ROWS = 16384
C_IN = 7
C_OUT = 6
BLK = 8192
GRID = N_ROWS // BLK


def _deepjet_body(x_ref, o_ref):
    x = x_ref[...]
    c = x[3:4, :]
    cvl = x[5:6, :]
    qg = x[6:7, :]
    t = c / cvl - c
    o_ref[...] = jnp.concatenate(
        [x[0:4, :], (1.0 - qg) * t, qg * t], axis=0
    )


@jax.jit
def _deepjet(xt):
    return pl.pallas_call(
        _deepjet_body,
        grid=(GRID,),
        in_specs=[pl.BlockSpec((C_IN, BLK), lambda i: (0, i))],
        out_specs=pl.BlockSpec((C_OUT, BLK), lambda i: (0, i)),
        out_shape=jax.ShapeDtypeStruct((C_OUT, N_ROWS), jnp.float32),
    )(xt)


def kernel(x):
    return _deepjet(x.T).T
